# baseline (speedup 1.0000x reference)
import functools

import jax
import jax.numpy as jnp
from jax import lax
from jax.experimental import pallas as pl
from jax.experimental.pallas import tpu as pltpu
from jax.experimental.pallas import tpu_sc as plsc

N = 10000
E = 160000
G = 100
HID = 128
LAT = 64
CUTOFF = 3.0

NC = 2
NS = 16
NW = NC * NS
EPAD = 163840
EPW = EPAD // NW
CH = 128
NCHUNK = EPW // CH
MW = 144
TE = 1024
TN = 1000
RB = 80
NBLK = N // RB

_f32 = jnp.float32


def _silu(v):
    return v * jax.nn.sigmoid(v)


PW = 16
TW = 256


GCH = 128
GNCH = EPW // GCH


def _sc_gather_body(a_h, b_h, dst3_h, src3_h, ag_h, bg_h,
                    idxd2, idxs2,
                    buf_a0, buf_a1, buf_b0, buf_b1,
                    sem_a0, sem_a1, sem_b0, sem_b1,
                    sem_w0, sem_w1, sem_v0, sem_v1):
    cid = lax.axis_index("c")
    sid = lax.axis_index("s")
    wid = cid * NS + sid
    base = wid * EPW
    pltpu.sync_copy(dst3_h.at[wid], idxd2)
    pltpu.sync_copy(src3_h.at[wid], idxs2)

    bufs = ((buf_a0, buf_b0, sem_a0, sem_b0, sem_w0, sem_v0),
            (buf_a1, buf_b1, sem_a1, sem_b1, sem_w1, sem_v1))

    def issue(c, s):
        ba, bb, sa, sb, _, _ = bufs[s]
        pltpu.async_copy(a_h.at[idxd2.at[c]], ba, sa)
        pltpu.async_copy(b_h.at[idxs2.at[c]], bb, sb)

    def step(c, s, first):
        ba, bb, sa, sb, sw, sv = bufs[s]
        pltpu.make_async_copy(a_h.at[idxd2.at[c]], ba, sa).wait()
        pltpu.make_async_copy(b_h.at[idxs2.at[c]], bb, sb).wait()
        sl = pl.ds(base + c * GCH, GCH)
        pltpu.async_copy(ba, ag_h.at[sl], sw)
        pltpu.async_copy(bb, bg_h.at[sl], sv)

        @pl.when(c + 2 < GNCH)
        def _():
            pltpu.make_async_copy(ba, ag_h.at[sl], sw).wait()
            pltpu.make_async_copy(bb, bg_h.at[sl], sv).wait()
            issue(c + 2, s)

    issue(0, 0)
    issue(1, 1)

    def pair(p, carry):
        step(2 * p, 0, p == 0)
        step(2 * p + 1, 1, p == 0)
        return carry
    lax.fori_loop(0, GNCH // 2, pair, 0)
    sl = pl.ds(base, GCH)
    pltpu.make_async_copy(buf_a0, ag_h.at[sl], sem_w0).wait()
    pltpu.make_async_copy(buf_b0, bg_h.at[sl], sem_v0).wait()
    pltpu.make_async_copy(buf_a1, ag_h.at[sl], sem_w1).wait()
    pltpu.make_async_copy(buf_b1, bg_h.at[sl], sem_v1).wait()


_sc_gather = pl.kernel(
    _sc_gather_body,
    mesh=plsc.VectorSubcoreMesh(core_axis_name="c", subcore_axis_name="s"),
    out_type=[
        jax.ShapeDtypeStruct((EPAD, HID), jnp.int32),
        jax.ShapeDtypeStruct((EPAD, HID), jnp.int32),
    ],
    scratch_types=[
        pltpu.VMEM((GNCH, GCH), jnp.int32), pltpu.VMEM((GNCH, GCH), jnp.int32),
        pltpu.VMEM((GCH, HID), jnp.int32), pltpu.VMEM((GCH, HID), jnp.int32),
        pltpu.VMEM((GCH, HID), jnp.int32), pltpu.VMEM((GCH, HID), jnp.int32),
        pltpu.SemaphoreType.DMA, pltpu.SemaphoreType.DMA,
        pltpu.SemaphoreType.DMA, pltpu.SemaphoreType.DMA,
        pltpu.SemaphoreType.DMA, pltpu.SemaphoreType.DMA,
        pltpu.SemaphoreType.DMA, pltpu.SemaphoreType.DMA,
    ],
)


def _sc_scatter_body(m_h, xv_h, dst3_h, pm0_h, pm1_h, px0_h, px1_h,
                     acc_s, mbuf0, mbuf1, idxd2, zbuf, sem_m0, sem_m1):
    cid = lax.axis_index("c")
    sid = lax.axis_index("s")
    wid = cid * NS + sid
    base = wid * EPW
    pltpu.sync_copy(dst3_h.at[wid], idxd2)

    def zrow(r, carry):
        for j in range(HID // 16):
            zbuf[r, pl.ds(j * 16, 16)] = jnp.zeros((16,), _f32)
        return carry
    lax.fori_loop(0, RB, zrow, 0)

    def phase(val_h, out0_h, out1_h):
        def zblk(k, carry):
            blk = sid + k * NS

            @pl.when(blk < NBLK)
            def _():
                pltpu.sync_copy(zbuf, acc_s.at[pl.ds(blk * RB, RB)])
            return carry
        lax.fori_loop(0, pl.cdiv(NBLK, NS), zblk, 0)
        plsc.subcore_barrier()

        bufs = ((mbuf0, sem_m0), (mbuf1, sem_m1))

        def issue(c, s):
            mb, sm = bufs[s]
            pltpu.async_copy(val_h.at[pl.ds(base + c * CH, CH)], mb, sm)

        def step(c, s):
            mb, sm = bufs[s]
            pltpu.make_async_copy(
                val_h.at[pl.ds(base, CH)], mb, sm).wait()

            @pl.when(c + 1 < NCHUNK)
            def _():
                issue(c + 1, 1 - s)
            pltpu.sync_copy(mb, acc_s.at[idxd2.at[c]], add=True)

        issue(0, 0)

        def pair(p, carry):
            step(2 * p, 0)
            step(2 * p + 1, 1)
            return carry
        lax.fori_loop(0, NCHUNK // 2, pair, 0)
        plsc.subcore_barrier()

        def wblk(k, carry):
            blk = sid + k * NS

            @pl.when(blk < NBLK)
            def _():
                sl = pl.ds(blk * RB, RB)

                @pl.when(cid == 0)
                def _():
                    pltpu.sync_copy(acc_s.at[sl], out0_h.at[sl])

                @pl.when(cid == 1)
                def _():
                    pltpu.sync_copy(acc_s.at[sl], out1_h.at[sl])
            return carry
        lax.fori_loop(0, pl.cdiv(NBLK, NS), wblk, 0)
        plsc.subcore_barrier()

    phase(m_h, pm0_h, pm1_h)
    phase(xv_h, px0_h, px1_h)


_sc_scatter = pl.kernel(
    _sc_scatter_body,
    mesh=plsc.VectorSubcoreMesh(core_axis_name="c", subcore_axis_name="s"),
    out_type=[
        jax.ShapeDtypeStruct((N, HID), _f32),
        jax.ShapeDtypeStruct((N, HID), _f32),
        jax.ShapeDtypeStruct((N, HID), _f32),
        jax.ShapeDtypeStruct((N, HID), _f32),
    ],
    scratch_types=[
        pltpu.VMEM_SHARED((N, HID), _f32),
        pltpu.VMEM((CH, HID), _f32), pltpu.VMEM((CH, HID), _f32),
        pltpu.VMEM((NCHUNK, CH), jnp.int32),
        pltpu.VMEM((RB, HID), _f32),
        pltpu.SemaphoreType.DMA, pltpu.SemaphoreType.DMA,
    ],
)


def _full(shape):
    return pl.BlockSpec(shape, lambda i: tuple(0 for _ in shape))


_bf16 = jnp.bfloat16
_HMASK = -65536


def _u16_bits(v):
    return lax.convert_element_type(
        lax.bitcast_convert_type(v.astype(_bf16), jnp.uint16), jnp.int32)


def _pack_row(v, xt):
    lo = _u16_bits(v[:, 0:64])
    hi = jnp.left_shift(_u16_bits(v[:, 64:HID]), 16)
    cr = _u16_bits(xt[:, 0:8])
    return jnp.concatenate(
        [lo | hi, cr, jnp.zeros((TN, HID - 72), jnp.int32)], axis=1)


def _store_ab(a_out, b_out, a, b, xt):
    a_out[...] = _pack_row(a, xt)
    b_out[...] = _pack_row(b, -xt)


def _enc_init_k(x_ref, xt_ref, wemb, bemb, wa, ba, wb, h_out, a_out, b_out):
    h0 = x_ref[...] * wemb[...] + bemb[...]
    h_out[...] = h0
    a = jnp.dot(h0, wa[...], preferred_element_type=_f32) + ba[...]
    b = jnp.dot(h0, wb[...], preferred_element_type=_f32)
    _store_ab(a_out, b_out, a, b, xt_ref[...])


def _edge_k(ag_ref, bg_ref, wd2, we2, be2, wx1, bx1, wx2, bx2, m_out, xv_out):
    i = pl.program_id(0)
    ag = ag_ref[...]
    bg = bg_ref[...]
    lo = (lax.bitcast_convert_type(jnp.left_shift(ag, 16), _f32)
          + lax.bitcast_convert_type(jnp.left_shift(bg, 16), _f32))
    hi = (lax.bitcast_convert_type(ag & _HMASK, _f32)
          + lax.bitcast_convert_type(bg & _HMASK, _f32))
    t = jnp.concatenate([lo[:, 0:64], hi[:, 0:64]], axis=1)
    dx = lo[:, 64:65]
    dy = lo[:, 65:66]
    dz = lo[:, 66:67]
    d2 = dx * dx + dy * dy + dz * dz
    u = _silu(t + d2 * wd2[...])
    v = _silu(jnp.dot(u.astype(_bf16), we2[...].astype(_bf16),
                      preferred_element_type=_f32) + be2[...])
    d = jnp.sqrt(d2 + 1e-8)
    w = 0.5 * (jnp.cos(jnp.pi * jnp.clip(d / CUTOFF, 0.0, 1.0)) + 1.0)
    gid = i * TE + lax.broadcasted_iota(jnp.int32, (TE, 1), 0)
    valid = (gid < E).astype(_f32)
    m = v * (w * valid)
    p = _silu(jnp.dot(m.astype(_bf16), wx1[...].astype(_bf16),
                      preferred_element_type=_f32) + bx1[...])
    cw = jnp.dot(p.astype(_bf16), wx2[...].astype(_bf16),
                 preferred_element_type=_f32) + bx2[...]
    inv = (cw * valid) / (d + 1.0)
    m_out[...] = m
    xv_out[...] = jnp.concatenate(
        [dx * inv, dy * inv, dz * inv, valid,
         jnp.zeros((TE, HID - 4), _f32)], axis=1)


def _node_k(pm0, pm1, px0, px1, h_ref, x_ref,
            wh1a, wh1b, bh1, wh2, bh2, wan, ban, wbn,
            h_out, a_out, b_out, x_out):
    agg = pm0[...] + pm1[...]
    ex = px0[:, 0:4] + px1[:, 0:4]
    deg = jnp.clip(ex[:, 3:4], 1.0, None)
    delta = jnp.concatenate(
        [ex[:, 0:3] / deg, jnp.zeros((TN, PW - 3), _f32)], axis=1)
    xn = x_ref[...] + delta
    x_out[...] = xn
    h = h_ref[...]
    hh = _silu(jnp.dot(h, wh1a[...], preferred_element_type=_f32)
               + jnp.dot(agg, wh1b[...], preferred_element_type=_f32) + bh1[...])
    hn = h + jnp.dot(hh, wh2[...], preferred_element_type=_f32) + bh2[...]
    h_out[...] = hn
    a = jnp.dot(hn, wan[...], preferred_element_type=_f32) + ban[...]
    b = jnp.dot(hn, wbn[...], preferred_element_type=_f32)
    _store_ab(a_out, b_out, a, b, xn)


def _node_fin_k(pm0, pm1, px0, px1, h_ref, x_ref,
                wh1a, wh1b, bh1, wh2, bh2, wo, bo,
                h_out, x_out):
    agg = pm0[...] + pm1[...]
    ex = px0[:, 0:4] + px1[:, 0:4]
    deg = jnp.clip(ex[:, 3:4], 1.0, None)
    delta = jnp.concatenate(
        [ex[:, 0:3] / deg, jnp.zeros((TN, PW - 3), _f32)], axis=1)
    x_out[...] = x_ref[...] + delta
    h = h_ref[...]
    hh = _silu(jnp.dot(h, wh1a[...], preferred_element_type=_f32)
               + jnp.dot(agg, wh1b[...], preferred_element_type=_f32) + bh1[...])
    hn = h + jnp.dot(hh, wh2[...], preferred_element_type=_f32) + bh2[...]
    h_out[...] = jnp.dot(hn, wo[...], preferred_element_type=_f32) + bo[...]


def _pool_k(h_ref, bcol, wmu, bmu, wlv, blv, eps_ref,
            mu_out, lv_out, z_out, acc):
    i = pl.program_id(0)

    @pl.when(i == 0)
    def _():
        acc[...] = jnp.zeros_like(acc)

    gi = lax.broadcasted_iota(jnp.int32, (TN, G), 1)
    mask = (gi == bcol[...]).astype(_f32)
    acc[...] += lax.dot_general(mask, h_ref[...], (((0,), (0,)), ((), ())),
                                preferred_element_type=_f32)

    @pl.when(i == pl.num_programs(0) - 1)
    def _():
        hp = acc[...]
        mu = jnp.dot(hp, wmu[...], preferred_element_type=_f32) + bmu[...]
        lv = jnp.dot(hp, wlv[...], preferred_element_type=_f32) + blv[...]
        mu_out[...] = mu
        lv_out[...] = lv
        z_out[...] = mu + eps_ref[...] * jnp.exp(0.5 * lv)


def _dec_init_k(z_ref, bcol, xt_ref, wemb, bemb, wa, ba, wb,
                h_out, a_out, b_out):
    gi = lax.broadcasted_iota(jnp.int32, (TN, G), 1)
    mask = (gi == bcol[...]).astype(_f32)
    zexp = jnp.dot(mask, z_ref[...], preferred_element_type=_f32)
    h0 = jnp.dot(zexp, wemb[...], preferred_element_type=_f32) + bemb[...]
    h_out[...] = h0
    a = jnp.dot(h0, wa[...], preferred_element_type=_f32) + ba[...]
    b = jnp.dot(h0, wb[...], preferred_element_type=_f32)
    _store_ab(a_out, b_out, a, b, xt_ref[...])


def _final_k(h_ref, bcol, wa1, ba1, wa2, ba2, we1, be1, we2, be2,
             atom_out, en_out, accp, accc):
    i = pl.program_id(0)
    h = h_ref[...]
    t1 = _silu(jnp.dot(h, wa1[...], preferred_element_type=_f32) + ba1[...])
    atom_out[...] = jax.nn.sigmoid(
        jnp.dot(t1, wa2[...], preferred_element_type=_f32) + ba2[...])

    @pl.when(i == 0)
    def _():
        accp[...] = jnp.zeros_like(accp)
        accc[...] = jnp.zeros_like(accc)

    gi = lax.broadcasted_iota(jnp.int32, (TN, G), 1)
    mask = (gi == bcol[...]).astype(_f32)
    accp[...] += lax.dot_general(mask, h, (((0,), (0,)), ((), ())),
                                 preferred_element_type=_f32)
    accc[...] += lax.dot_general(mask, jnp.ones((TN, 1), _f32),
                                 (((0,), (0,)), ((), ())),
                                 preferred_element_type=_f32)

    @pl.when(i == pl.num_programs(0) - 1)
    def _():
        pooled = accp[...] / jnp.clip(accc[...], 1.0, None)
        e1o = _silu(jnp.dot(pooled, we1[...], preferred_element_type=_f32) + be1[...])
        en_out[...] = jnp.dot(e1o, we2[...], preferred_element_type=_f32) + be2[...]


_NGRID = N // TN

_spec_nh = pl.BlockSpec((TN, HID), lambda i: (i, 0))
_spec_n1 = pl.BlockSpec((TN, 1), lambda i: (i, 0))
_spec_nx = pl.BlockSpec((TN, PW), lambda i: (i, 0))
_spec_eh = pl.BlockSpec((TE, HID), lambda i: (i, 0))
_spec_ex = pl.BlockSpec((TE, PW), lambda i: (i, 0))
_spec_emw = pl.BlockSpec((TE, MW), lambda i: (i, 0))
_spec_pmw = pl.BlockSpec((TN, MW), lambda i: (i, 0))

_sd = jax.ShapeDtypeStruct


_spec_ntw = pl.BlockSpec((TN, HID), lambda i: (i, 0))
_ab_sd = _sd((N, HID), jnp.int32)


def _enc_init(x1, xtab, wemb, bemb, wa, ba, wb):
    return pl.pallas_call(
        _enc_init_k, grid=(_NGRID,),
        in_specs=[_spec_n1, _spec_nx, _full((1, HID)), _full((1, HID)),
                  _full((HID, HID)), _full((1, HID)), _full((HID, HID))],
        out_specs=[_spec_nh, _spec_ntw, _spec_ntw],
        out_shape=[_sd((N, HID), _f32), _ab_sd, _ab_sd],
    )(x1, xtab, wemb, bemb, wa, ba, wb)


_spec_etex = pl.BlockSpec((TE, HID), lambda i: (i, 0))


def _edge(ag, bg, wd2, we2, be2, wx1, bx1, wx2, bx2):
    return pl.pallas_call(
        _edge_k, grid=(EPAD // TE,),
        in_specs=[_spec_etex, _spec_etex,
                  _full((1, HID)), _full((HID, HID)), _full((1, HID)),
                  _full((HID, HID)), _full((1, HID)), _full((HID, 1)),
                  _full((1, 1))],
        out_specs=[_spec_eh, _spec_eh],
        out_shape=[_sd((EPAD, HID), _f32), _sd((EPAD, HID), _f32)],
    )(ag, bg, wd2, we2, be2, wx1, bx1, wx2, bx2)


def _node(pm0, pm1, px0, px1, h, xtab,
          wh1a, wh1b, bh1, wh2, bh2, wan, ban, wbn):
    return pl.pallas_call(
        _node_k, grid=(_NGRID,),
        in_specs=[_spec_nh, _spec_nh, _spec_nh, _spec_nh, _spec_nh, _spec_nx,
                  _full((HID, HID)), _full((HID, HID)), _full((1, HID)),
                  _full((HID, HID)), _full((1, HID)),
                  _full((HID, HID)), _full((1, HID)), _full((HID, HID))],
        out_specs=[_spec_nh, _spec_ntw, _spec_ntw, _spec_nx],
        out_shape=[_sd((N, HID), _f32), _ab_sd, _ab_sd,
                   _sd((N, PW), _f32)],
    )(pm0, pm1, px0, px1, h, xtab, wh1a, wh1b, bh1, wh2, bh2, wan, ban, wbn)


def _node_fin(pm0, pm1, px0, px1, h, xtab,
              wh1a, wh1b, bh1, wh2, bh2, wo, bo):
    return pl.pallas_call(
        _node_fin_k, grid=(_NGRID,),
        in_specs=[_spec_nh, _spec_nh, _spec_nh, _spec_nh, _spec_nh, _spec_nx,
                  _full((HID, HID)), _full((HID, HID)), _full((1, HID)),
                  _full((HID, HID)), _full((1, HID)),
                  _full((HID, HID)), _full((1, HID))],
        out_specs=[_spec_nh, _spec_nx],
        out_shape=[_sd((N, HID), _f32), _sd((N, PW), _f32)],
    )(pm0, pm1, px0, px1, h, xtab, wh1a, wh1b, bh1, wh2, bh2, wo, bo)


def _pool(h, bcol, wmu, bmu, wlv, blv, eps):
    return pl.pallas_call(
        _pool_k, grid=(_NGRID,),
        in_specs=[_spec_nh, _spec_n1,
                  _full((HID, LAT)), _full((1, LAT)),
                  _full((HID, LAT)), _full((1, LAT)), _full((G, LAT))],
        out_specs=[_full((G, LAT))] * 3,
        out_shape=[_sd((G, LAT), _f32)] * 3,
        scratch_shapes=[pltpu.VMEM((G, HID), _f32)],
    )(h, bcol, wmu, bmu, wlv, blv, eps)


def _dec_init(z, bcol, xtab, wemb, bemb, wa, ba, wb):
    return pl.pallas_call(
        _dec_init_k, grid=(_NGRID,),
        in_specs=[_full((G, LAT)), _spec_n1, _spec_nx,
                  _full((LAT, HID)), _full((1, HID)),
                  _full((HID, HID)), _full((1, HID)), _full((HID, HID))],
        out_specs=[_spec_nh, _spec_ntw, _spec_ntw],
        out_shape=[_sd((N, HID), _f32), _ab_sd, _ab_sd],
    )(z, bcol, xtab, wemb, bemb, wa, ba, wb)


def _final(h, bcol, wa1, ba1, wa2, ba2, we1, be1, we2, be2):
    return pl.pallas_call(
        _final_k, grid=(_NGRID,),
        in_specs=[_spec_nh, _spec_n1,
                  _full((HID, HID)), _full((1, HID)), _full((HID, 1)),
                  _full((1, 1)),
                  _full((HID, HID // 2)), _full((1, HID // 2)),
                  _full((HID // 2, 1)), _full((1, 1))],
        out_specs=[_spec_n1, _full((G, 1))],
        out_shape=[_sd((N, 1), _f32), _sd((G, 1), _f32)],
        scratch_shapes=[pltpu.VMEM((G, HID), _f32), pltpu.VMEM((G, 1), _f32)],
    )(h, bcol, wa1, ba1, wa2, ba2, we1, be1, we2, be2)


def _row(v):
    return v.reshape(1, -1)


def _egnn_layers(layers, h, a, b, xtab, srcg, dstg, dsts, final_w, final_b):
    n_layers = len(layers)
    for li, lp in enumerate(layers):
        ag, bg = _sc_gather(a, b, dstg, srcg)
        w1 = lp["e1"]["W"]
        m, xv = _edge(
            ag, bg,
            _row(w1[2 * HID]), lp["e2"]["W"], _row(lp["e2"]["b"]),
            lp["x1"]["W"], _row(lp["x1"]["b"]),
            lp["x2"]["W"], _row(lp["x2"]["b"]))
        pm0, pm1, px0, px1 = _sc_scatter(m, xv, dsts)
        wh1 = lp["h1"]["W"]
        if li + 1 < n_layers:
            nx = layers[li + 1]["e1"]["W"]
            nb = layers[li + 1]["e1"]["b"]
            h, a, b, xtab = _node(
                pm0, pm1, px0, px1, h, xtab,
                wh1[:HID], wh1[HID:], _row(lp["h1"]["b"]),
                lp["h2"]["W"], _row(lp["h2"]["b"]),
                nx[:HID], _row(nb), nx[HID:2 * HID])
        else:
            hout, xtab = _node_fin(
                pm0, pm1, px0, px1, h, xtab,
                wh1[:HID], wh1[HID:], _row(lp["h1"]["b"]),
                lp["h2"]["W"], _row(lp["h2"]["b"]),
                final_w, _row(final_b))
    return hout, xtab


def kernel(x, pos, batch, edge_index, params):
    src = edge_index[0]
    dst = edge_index[1]
    pad = jnp.zeros((EPAD - E,), dst.dtype)
    srcg = jnp.concatenate([src, pad]).reshape(NW, GNCH, GCH)
    dstp = jnp.concatenate([dst, pad])
    dstg = dstp.reshape(NW, GNCH, GCH)
    dsts = dstp.reshape(NW, NCHUNK, CH)
    bcol = batch.astype(jnp.int32).reshape(N, 1)

    eps = jax.random.normal(jax.random.key(42), (G, LAT), _f32)
    noise = jax.random.normal(jax.random.key(43), (N, 3), _f32) * 0.01
    cinit = params["coord_init"] + noise

    enc = params["enc"]
    e0 = enc["layers"][0]["e1"]
    xtab0 = jnp.concatenate([pos, jnp.zeros((N, PW - 3), _f32)], axis=1)
    h, a, b = _enc_init(
        x, xtab0, enc["emb_in"]["W"].reshape(1, HID), _row(enc["emb_in"]["b"]),
        e0["W"][:HID], _row(e0["b"]), e0["W"][HID:2 * HID])
    h_enc, _ = _egnn_layers(
        enc["layers"], h, a, b, xtab0, srcg, dstg, dsts,
        enc["emb_out"]["W"], enc["emb_out"]["b"])

    mu, logvar, z = _pool(
        h_enc, bcol,
        params["fc_mu"]["W"], _row(params["fc_mu"]["b"]),
        params["fc_logvar"]["W"], _row(params["fc_logvar"]["b"]), eps)

    dec = params["dec"]
    d0 = dec["layers"][0]["e1"]
    ctab0 = jnp.concatenate([cinit, jnp.zeros((N, PW - 3), _f32)], axis=1)
    hd, ad, bd = _dec_init(
        z, bcol, ctab0, dec["emb_in"]["W"], _row(dec["emb_in"]["b"]),
        d0["W"][:HID], _row(d0["b"]), d0["W"][HID:2 * HID])
    h_dec, ctab = _egnn_layers(
        dec["layers"], hd, ad, bd, ctab0, srcg, dstg, dsts,
        dec["emb_out"]["W"], dec["emb_out"]["b"])

    atom_pred, energy_pred = _final(
        h_dec, bcol,
        params["atom1"]["W"], _row(params["atom1"]["b"]),
        params["atom2"]["W"], _row(params["atom2"]["b"]),
        params["en1"]["W"], _row(params["en1"]["b"]),
        params["en2"]["W"], _row(params["en2"]["b"]))

    coords_out = ctab[:, 0:3]
    return (atom_pred, coords_out, energy_pred, mu, logvar)


# speedup vs baseline: 1.1593x; 1.1593x over previous
import functools

import jax
import jax.numpy as jnp
from jax import lax
from jax.experimental import pallas as pl
from jax.experimental.pallas import tpu as pltpu
from jax.experimental.pallas import tpu_sc as plsc

N = 10000
E = 160000
G = 100
HID = 128
LAT = 64
CUTOFF = 3.0

NC = 2
NS = 16
NW = NC * NS
EPAD = 163840
EPW = EPAD // NW
CH = 128
NCHUNK = EPW // CH
MW = 144
TE = 1024
TN = 1000
RB = 80
NBLK = N // RB

_f32 = jnp.float32


def _silu(v):
    return v * jax.nn.sigmoid(v)


PW = 16
TW = 256


GCH = 128
GNCH = EPW // GCH


def _sc_gather_body(a_h, b_h, dst3_h, src3_h, ag_h, bg_h,
                    idxd2, idxs2,
                    buf_a0, buf_a1, buf_b0, buf_b1,
                    sem_a0, sem_a1, sem_b0, sem_b1,
                    sem_w0, sem_w1, sem_v0, sem_v1):
    cid = lax.axis_index("c")
    sid = lax.axis_index("s")
    wid = cid * NS + sid
    base = wid * EPW
    pltpu.sync_copy(dst3_h.at[wid], idxd2)
    pltpu.sync_copy(src3_h.at[wid], idxs2)

    bufs = ((buf_a0, buf_b0, sem_a0, sem_b0, sem_w0, sem_v0),
            (buf_a1, buf_b1, sem_a1, sem_b1, sem_w1, sem_v1))

    def issue(c, s):
        ba, bb, sa, sb, _, _ = bufs[s]
        pltpu.async_copy(a_h.at[idxd2.at[c]], ba, sa)
        pltpu.async_copy(b_h.at[idxs2.at[c]], bb, sb)

    def step(c, s, first):
        ba, bb, sa, sb, sw, sv = bufs[s]
        pltpu.make_async_copy(a_h.at[idxd2.at[c]], ba, sa).wait()
        pltpu.make_async_copy(b_h.at[idxs2.at[c]], bb, sb).wait()
        sl = pl.ds(base + c * GCH, GCH)
        pltpu.async_copy(ba, ag_h.at[sl], sw)
        pltpu.async_copy(bb, bg_h.at[sl], sv)

        @pl.when(c + 2 < GNCH)
        def _():
            pltpu.make_async_copy(ba, ag_h.at[sl], sw).wait()
            pltpu.make_async_copy(bb, bg_h.at[sl], sv).wait()
            issue(c + 2, s)

    issue(0, 0)
    issue(1, 1)

    def pair(p, carry):
        step(2 * p, 0, p == 0)
        step(2 * p + 1, 1, p == 0)
        return carry
    lax.fori_loop(0, GNCH // 2, pair, 0)
    sl = pl.ds(base, GCH)
    pltpu.make_async_copy(buf_a0, ag_h.at[sl], sem_w0).wait()
    pltpu.make_async_copy(buf_b0, bg_h.at[sl], sem_v0).wait()
    pltpu.make_async_copy(buf_a1, ag_h.at[sl], sem_w1).wait()
    pltpu.make_async_copy(buf_b1, bg_h.at[sl], sem_v1).wait()


_sc_gather = pl.kernel(
    _sc_gather_body,
    mesh=plsc.VectorSubcoreMesh(core_axis_name="c", subcore_axis_name="s"),
    out_type=[
        jax.ShapeDtypeStruct((EPAD, HID), jnp.int32),
        jax.ShapeDtypeStruct((EPAD, HID), jnp.int32),
    ],
    scratch_types=[
        pltpu.VMEM((GNCH, GCH), jnp.int32), pltpu.VMEM((GNCH, GCH), jnp.int32),
        pltpu.VMEM((GCH, HID), jnp.int32), pltpu.VMEM((GCH, HID), jnp.int32),
        pltpu.VMEM((GCH, HID), jnp.int32), pltpu.VMEM((GCH, HID), jnp.int32),
        pltpu.SemaphoreType.DMA, pltpu.SemaphoreType.DMA,
        pltpu.SemaphoreType.DMA, pltpu.SemaphoreType.DMA,
        pltpu.SemaphoreType.DMA, pltpu.SemaphoreType.DMA,
        pltpu.SemaphoreType.DMA, pltpu.SemaphoreType.DMA,
    ],
)


def _sc_scatter_body(m_h, xv_h, dst3_h, pm0_h, pm1_h, px0_h, px1_h,
                     acc_s, mbuf0, mbuf1, idxd2, zbuf, sem_m0, sem_m1):
    cid = lax.axis_index("c")
    sid = lax.axis_index("s")
    wid = cid * NS + sid
    base = wid * EPW
    pltpu.sync_copy(dst3_h.at[wid], idxd2)

    def zrow(r, carry):
        for j in range(HID // 16):
            zbuf[r, pl.ds(j * 16, 16)] = jnp.zeros((16,), _f32)
        return carry
    lax.fori_loop(0, RB, zrow, 0)

    def phase(val_h, out0_h, out1_h):
        def zblk(k, carry):
            blk = sid + k * NS

            @pl.when(blk < NBLK)
            def _():
                pltpu.sync_copy(zbuf, acc_s.at[pl.ds(blk * RB, RB)])
            return carry
        lax.fori_loop(0, pl.cdiv(NBLK, NS), zblk, 0)
        plsc.subcore_barrier()

        bufs = ((mbuf0, sem_m0), (mbuf1, sem_m1))

        def issue(c, s):
            mb, sm = bufs[s]
            pltpu.async_copy(val_h.at[pl.ds(base + c * CH, CH)], mb, sm)

        def step(c, s):
            mb, sm = bufs[s]
            pltpu.make_async_copy(
                val_h.at[pl.ds(base, CH)], mb, sm).wait()

            @pl.when(c + 1 < NCHUNK)
            def _():
                issue(c + 1, 1 - s)
            pltpu.sync_copy(mb, acc_s.at[idxd2.at[c]], add=True)

        issue(0, 0)

        def pair(p, carry):
            step(2 * p, 0)
            step(2 * p + 1, 1)
            return carry
        lax.fori_loop(0, NCHUNK // 2, pair, 0)
        plsc.subcore_barrier()

        def wblk(k, carry):
            blk = sid + k * NS

            @pl.when(blk < NBLK)
            def _():
                sl = pl.ds(blk * RB, RB)

                @pl.when(cid == 0)
                def _():
                    pltpu.sync_copy(acc_s.at[sl], out0_h.at[sl])

                @pl.when(cid == 1)
                def _():
                    pltpu.sync_copy(acc_s.at[sl], out1_h.at[sl])
            return carry
        lax.fori_loop(0, pl.cdiv(NBLK, NS), wblk, 0)
        plsc.subcore_barrier()

    phase(m_h, pm0_h, pm1_h)
    phase(xv_h, px0_h, px1_h)


_sc_scatter = pl.kernel(
    _sc_scatter_body,
    mesh=plsc.VectorSubcoreMesh(core_axis_name="c", subcore_axis_name="s"),
    out_type=[
        jax.ShapeDtypeStruct((N, HID), _f32),
        jax.ShapeDtypeStruct((N, HID), _f32),
        jax.ShapeDtypeStruct((N, HID), _f32),
        jax.ShapeDtypeStruct((N, HID), _f32),
    ],
    scratch_types=[
        pltpu.VMEM_SHARED((N, HID), _f32),
        pltpu.VMEM((CH, HID), _f32), pltpu.VMEM((CH, HID), _f32),
        pltpu.VMEM((NCHUNK, CH), jnp.int32),
        pltpu.VMEM((RB, HID), _f32),
        pltpu.SemaphoreType.DMA, pltpu.SemaphoreType.DMA,
    ],
)


def _full(shape):
    return pl.BlockSpec(shape, lambda i: tuple(0 for _ in shape))


_bf16 = jnp.bfloat16
_HMASK = -65536


def _u16_bits(v):
    return lax.convert_element_type(
        lax.bitcast_convert_type(v.astype(_bf16), jnp.uint16), jnp.int32)


def _pack_row(v, xt):
    lo = _u16_bits(v[:, 0:64])
    hi = jnp.left_shift(_u16_bits(v[:, 64:HID]), 16)
    cr = _u16_bits(xt[:, 0:8])
    return jnp.concatenate(
        [lo | hi, cr, jnp.zeros((TN, HID - 72), jnp.int32)], axis=1)


def _store_ab(a_out, b_out, a, b, xt):
    a_out[...] = _pack_row(a, xt)
    b_out[...] = _pack_row(b, -xt)


def _enc_init_k(x_ref, xt_ref, wemb, bemb, wa, ba, wb, h_out, a_out, b_out):
    h0 = x_ref[...] * wemb[...] + bemb[...]
    h_out[...] = h0
    a = jnp.dot(h0, wa[...], preferred_element_type=_f32) + ba[...]
    b = jnp.dot(h0, wb[...], preferred_element_type=_f32)
    _store_ab(a_out, b_out, a, b, xt_ref[...])


def _edge_k(ag_ref, bg_ref, wd2, we2, be2, wx1, bx1, wx2, bx2, m_out, xv_out):
    i = pl.program_id(0)
    ag = ag_ref[...]
    bg = bg_ref[...]
    lo = (lax.bitcast_convert_type(jnp.left_shift(ag, 16), _f32)
          + lax.bitcast_convert_type(jnp.left_shift(bg, 16), _f32))
    hi = (lax.bitcast_convert_type(ag & _HMASK, _f32)
          + lax.bitcast_convert_type(bg & _HMASK, _f32))
    t = jnp.concatenate([lo[:, 0:64], hi[:, 0:64]], axis=1)

    def lane(v):
        return jnp.transpose(v, (1, 0))

    def col(v):
        return jnp.transpose(v, (1, 0))

    dxl = lane(lo[:, 64:65])
    dyl = lane(lo[:, 65:66])
    dzl = lane(lo[:, 66:67])
    d2l = dxl * dxl + dyl * dyl + dzl * dzl
    dl = jnp.sqrt(d2l + 1e-8)
    wl = 0.5 * (jnp.cos(jnp.pi * jnp.clip(dl / CUTOFF, 0.0, 1.0)) + 1.0)
    gidl = i * TE + lax.broadcasted_iota(jnp.int32, (1, TE), 1)
    validl = (gidl < E).astype(_f32)
    wvl = wl * validl

    u = _silu(t + col(d2l) * wd2[...])
    v = _silu(jnp.dot(u.astype(_bf16), we2[...].astype(_bf16),
                      preferred_element_type=_f32) + be2[...])
    m = v * col(wvl)
    p = _silu(jnp.dot(m.astype(_bf16), wx1[...].astype(_bf16),
                      preferred_element_type=_f32) + bx1[...])
    cw = jnp.dot(p.astype(_bf16), wx2[...].astype(_bf16),
                 preferred_element_type=_f32) + bx2[...]
    invl = lane(cw) * validl / (dl + 1.0)
    m_out[...] = m
    xv_out[...] = jnp.concatenate(
        [col(dxl * invl), col(dyl * invl), col(dzl * invl), col(validl),
         jnp.zeros((TE, HID - 4), _f32)], axis=1)


def _node_k(pm0, pm1, px0, px1, h_ref, x_ref,
            wh1a, wh1b, bh1, wh2, bh2, wan, ban, wbn,
            h_out, a_out, b_out, x_out):
    agg = pm0[...] + pm1[...]
    ex = px0[:, 0:4] + px1[:, 0:4]
    deg = jnp.clip(ex[:, 3:4], 1.0, None)
    delta = jnp.concatenate(
        [ex[:, 0:3] / deg, jnp.zeros((TN, PW - 3), _f32)], axis=1)
    xn = x_ref[...] + delta
    x_out[...] = xn
    h = h_ref[...]
    hh = _silu(jnp.dot(h, wh1a[...], preferred_element_type=_f32)
               + jnp.dot(agg, wh1b[...], preferred_element_type=_f32) + bh1[...])
    hn = h + jnp.dot(hh, wh2[...], preferred_element_type=_f32) + bh2[...]
    h_out[...] = hn
    a = jnp.dot(hn, wan[...], preferred_element_type=_f32) + ban[...]
    b = jnp.dot(hn, wbn[...], preferred_element_type=_f32)
    _store_ab(a_out, b_out, a, b, xn)


def _node_fin_k(pm0, pm1, px0, px1, h_ref, x_ref,
                wh1a, wh1b, bh1, wh2, bh2, wo, bo,
                h_out, x_out):
    agg = pm0[...] + pm1[...]
    ex = px0[:, 0:4] + px1[:, 0:4]
    deg = jnp.clip(ex[:, 3:4], 1.0, None)
    delta = jnp.concatenate(
        [ex[:, 0:3] / deg, jnp.zeros((TN, PW - 3), _f32)], axis=1)
    x_out[...] = x_ref[...] + delta
    h = h_ref[...]
    hh = _silu(jnp.dot(h, wh1a[...], preferred_element_type=_f32)
               + jnp.dot(agg, wh1b[...], preferred_element_type=_f32) + bh1[...])
    hn = h + jnp.dot(hh, wh2[...], preferred_element_type=_f32) + bh2[...]
    h_out[...] = jnp.dot(hn, wo[...], preferred_element_type=_f32) + bo[...]


def _pool_k(h_ref, bcol, wmu, bmu, wlv, blv, eps_ref,
            mu_out, lv_out, z_out, acc):
    i = pl.program_id(0)

    @pl.when(i == 0)
    def _():
        acc[...] = jnp.zeros_like(acc)

    gi = lax.broadcasted_iota(jnp.int32, (TN, G), 1)
    mask = (gi == bcol[...]).astype(_f32)
    acc[...] += lax.dot_general(mask, h_ref[...], (((0,), (0,)), ((), ())),
                                preferred_element_type=_f32)

    @pl.when(i == pl.num_programs(0) - 1)
    def _():
        hp = acc[...]
        mu = jnp.dot(hp, wmu[...], preferred_element_type=_f32) + bmu[...]
        lv = jnp.dot(hp, wlv[...], preferred_element_type=_f32) + blv[...]
        mu_out[...] = mu
        lv_out[...] = lv
        z_out[...] = mu + eps_ref[...] * jnp.exp(0.5 * lv)


def _dec_init_k(z_ref, bcol, xt_ref, wemb, bemb, wa, ba, wb,
                h_out, a_out, b_out):
    gi = lax.broadcasted_iota(jnp.int32, (TN, G), 1)
    mask = (gi == bcol[...]).astype(_f32)
    zexp = jnp.dot(mask, z_ref[...], preferred_element_type=_f32)
    h0 = jnp.dot(zexp, wemb[...], preferred_element_type=_f32) + bemb[...]
    h_out[...] = h0
    a = jnp.dot(h0, wa[...], preferred_element_type=_f32) + ba[...]
    b = jnp.dot(h0, wb[...], preferred_element_type=_f32)
    _store_ab(a_out, b_out, a, b, xt_ref[...])


def _final_k(h_ref, bcol, wa1, ba1, wa2, ba2, we1, be1, we2, be2,
             atom_out, en_out, accp, accc):
    i = pl.program_id(0)
    h = h_ref[...]
    t1 = _silu(jnp.dot(h, wa1[...], preferred_element_type=_f32) + ba1[...])
    atom_out[...] = jax.nn.sigmoid(
        jnp.dot(t1, wa2[...], preferred_element_type=_f32) + ba2[...])

    @pl.when(i == 0)
    def _():
        accp[...] = jnp.zeros_like(accp)
        accc[...] = jnp.zeros_like(accc)

    gi = lax.broadcasted_iota(jnp.int32, (TN, G), 1)
    mask = (gi == bcol[...]).astype(_f32)
    accp[...] += lax.dot_general(mask, h, (((0,), (0,)), ((), ())),
                                 preferred_element_type=_f32)
    accc[...] += lax.dot_general(mask, jnp.ones((TN, 1), _f32),
                                 (((0,), (0,)), ((), ())),
                                 preferred_element_type=_f32)

    @pl.when(i == pl.num_programs(0) - 1)
    def _():
        pooled = accp[...] / jnp.clip(accc[...], 1.0, None)
        e1o = _silu(jnp.dot(pooled, we1[...], preferred_element_type=_f32) + be1[...])
        en_out[...] = jnp.dot(e1o, we2[...], preferred_element_type=_f32) + be2[...]


_NGRID = N // TN

_spec_nh = pl.BlockSpec((TN, HID), lambda i: (i, 0))
_spec_n1 = pl.BlockSpec((TN, 1), lambda i: (i, 0))
_spec_nx = pl.BlockSpec((TN, PW), lambda i: (i, 0))
_spec_eh = pl.BlockSpec((TE, HID), lambda i: (i, 0))
_spec_ex = pl.BlockSpec((TE, PW), lambda i: (i, 0))
_spec_emw = pl.BlockSpec((TE, MW), lambda i: (i, 0))
_spec_pmw = pl.BlockSpec((TN, MW), lambda i: (i, 0))

_sd = jax.ShapeDtypeStruct


_spec_ntw = pl.BlockSpec((TN, HID), lambda i: (i, 0))
_ab_sd = _sd((N, HID), jnp.int32)


def _enc_init(x1, xtab, wemb, bemb, wa, ba, wb):
    return pl.pallas_call(
        _enc_init_k, grid=(_NGRID,),
        in_specs=[_spec_n1, _spec_nx, _full((1, HID)), _full((1, HID)),
                  _full((HID, HID)), _full((1, HID)), _full((HID, HID))],
        out_specs=[_spec_nh, _spec_ntw, _spec_ntw],
        out_shape=[_sd((N, HID), _f32), _ab_sd, _ab_sd],
    )(x1, xtab, wemb, bemb, wa, ba, wb)


_spec_etex = pl.BlockSpec((TE, HID), lambda i: (i, 0))


def _edge(ag, bg, wd2, we2, be2, wx1, bx1, wx2, bx2):
    return pl.pallas_call(
        _edge_k, grid=(EPAD // TE,),
        in_specs=[_spec_etex, _spec_etex,
                  _full((1, HID)), _full((HID, HID)), _full((1, HID)),
                  _full((HID, HID)), _full((1, HID)), _full((HID, 1)),
                  _full((1, 1))],
        out_specs=[_spec_eh, _spec_eh],
        out_shape=[_sd((EPAD, HID), _f32), _sd((EPAD, HID), _f32)],
    )(ag, bg, wd2, we2, be2, wx1, bx1, wx2, bx2)


def _node(pm0, pm1, px0, px1, h, xtab,
          wh1a, wh1b, bh1, wh2, bh2, wan, ban, wbn):
    return pl.pallas_call(
        _node_k, grid=(_NGRID,),
        in_specs=[_spec_nh, _spec_nh, _spec_nh, _spec_nh, _spec_nh, _spec_nx,
                  _full((HID, HID)), _full((HID, HID)), _full((1, HID)),
                  _full((HID, HID)), _full((1, HID)),
                  _full((HID, HID)), _full((1, HID)), _full((HID, HID))],
        out_specs=[_spec_nh, _spec_ntw, _spec_ntw, _spec_nx],
        out_shape=[_sd((N, HID), _f32), _ab_sd, _ab_sd,
                   _sd((N, PW), _f32)],
    )(pm0, pm1, px0, px1, h, xtab, wh1a, wh1b, bh1, wh2, bh2, wan, ban, wbn)


def _node_fin(pm0, pm1, px0, px1, h, xtab,
              wh1a, wh1b, bh1, wh2, bh2, wo, bo):
    return pl.pallas_call(
        _node_fin_k, grid=(_NGRID,),
        in_specs=[_spec_nh, _spec_nh, _spec_nh, _spec_nh, _spec_nh, _spec_nx,
                  _full((HID, HID)), _full((HID, HID)), _full((1, HID)),
                  _full((HID, HID)), _full((1, HID)),
                  _full((HID, HID)), _full((1, HID))],
        out_specs=[_spec_nh, _spec_nx],
        out_shape=[_sd((N, HID), _f32), _sd((N, PW), _f32)],
    )(pm0, pm1, px0, px1, h, xtab, wh1a, wh1b, bh1, wh2, bh2, wo, bo)


def _pool(h, bcol, wmu, bmu, wlv, blv, eps):
    return pl.pallas_call(
        _pool_k, grid=(_NGRID,),
        in_specs=[_spec_nh, _spec_n1,
                  _full((HID, LAT)), _full((1, LAT)),
                  _full((HID, LAT)), _full((1, LAT)), _full((G, LAT))],
        out_specs=[_full((G, LAT))] * 3,
        out_shape=[_sd((G, LAT), _f32)] * 3,
        scratch_shapes=[pltpu.VMEM((G, HID), _f32)],
    )(h, bcol, wmu, bmu, wlv, blv, eps)


def _dec_init(z, bcol, xtab, wemb, bemb, wa, ba, wb):
    return pl.pallas_call(
        _dec_init_k, grid=(_NGRID,),
        in_specs=[_full((G, LAT)), _spec_n1, _spec_nx,
                  _full((LAT, HID)), _full((1, HID)),
                  _full((HID, HID)), _full((1, HID)), _full((HID, HID))],
        out_specs=[_spec_nh, _spec_ntw, _spec_ntw],
        out_shape=[_sd((N, HID), _f32), _ab_sd, _ab_sd],
    )(z, bcol, xtab, wemb, bemb, wa, ba, wb)


def _final(h, bcol, wa1, ba1, wa2, ba2, we1, be1, we2, be2):
    return pl.pallas_call(
        _final_k, grid=(_NGRID,),
        in_specs=[_spec_nh, _spec_n1,
                  _full((HID, HID)), _full((1, HID)), _full((HID, 1)),
                  _full((1, 1)),
                  _full((HID, HID // 2)), _full((1, HID // 2)),
                  _full((HID // 2, 1)), _full((1, 1))],
        out_specs=[_spec_n1, _full((G, 1))],
        out_shape=[_sd((N, 1), _f32), _sd((G, 1), _f32)],
        scratch_shapes=[pltpu.VMEM((G, HID), _f32), pltpu.VMEM((G, 1), _f32)],
    )(h, bcol, wa1, ba1, wa2, ba2, we1, be1, we2, be2)


def _row(v):
    return v.reshape(1, -1)


def _egnn_layers(layers, h, a, b, xtab, srcg, dstg, dsts, final_w, final_b):
    n_layers = len(layers)
    for li, lp in enumerate(layers):
        ag, bg = _sc_gather(a, b, dstg, srcg)
        w1 = lp["e1"]["W"]
        m, xv = _edge(
            ag, bg,
            _row(w1[2 * HID]), lp["e2"]["W"], _row(lp["e2"]["b"]),
            lp["x1"]["W"], _row(lp["x1"]["b"]),
            lp["x2"]["W"], _row(lp["x2"]["b"]))
        pm0, pm1, px0, px1 = _sc_scatter(m, xv, dsts)
        wh1 = lp["h1"]["W"]
        if li + 1 < n_layers:
            nx = layers[li + 1]["e1"]["W"]
            nb = layers[li + 1]["e1"]["b"]
            h, a, b, xtab = _node(
                pm0, pm1, px0, px1, h, xtab,
                wh1[:HID], wh1[HID:], _row(lp["h1"]["b"]),
                lp["h2"]["W"], _row(lp["h2"]["b"]),
                nx[:HID], _row(nb), nx[HID:2 * HID])
        else:
            hout, xtab = _node_fin(
                pm0, pm1, px0, px1, h, xtab,
                wh1[:HID], wh1[HID:], _row(lp["h1"]["b"]),
                lp["h2"]["W"], _row(lp["h2"]["b"]),
                final_w, _row(final_b))
    return hout, xtab


def kernel(x, pos, batch, edge_index, params):
    src = edge_index[0]
    dst = edge_index[1]
    pad = jnp.zeros((EPAD - E,), dst.dtype)
    srcg = jnp.concatenate([src, pad]).reshape(NW, GNCH, GCH)
    dstp = jnp.concatenate([dst, pad])
    dstg = dstp.reshape(NW, GNCH, GCH)
    dsts = dstp.reshape(NW, NCHUNK, CH)
    bcol = batch.astype(jnp.int32).reshape(N, 1)

    eps = jax.random.normal(jax.random.key(42), (G, LAT), _f32)
    noise = jax.random.normal(jax.random.key(43), (N, 3), _f32) * 0.01
    cinit = params["coord_init"] + noise

    enc = params["enc"]
    e0 = enc["layers"][0]["e1"]
    xtab0 = jnp.concatenate([pos, jnp.zeros((N, PW - 3), _f32)], axis=1)
    h, a, b = _enc_init(
        x, xtab0, enc["emb_in"]["W"].reshape(1, HID), _row(enc["emb_in"]["b"]),
        e0["W"][:HID], _row(e0["b"]), e0["W"][HID:2 * HID])
    h_enc, _ = _egnn_layers(
        enc["layers"], h, a, b, xtab0, srcg, dstg, dsts,
        enc["emb_out"]["W"], enc["emb_out"]["b"])

    mu, logvar, z = _pool(
        h_enc, bcol,
        params["fc_mu"]["W"], _row(params["fc_mu"]["b"]),
        params["fc_logvar"]["W"], _row(params["fc_logvar"]["b"]), eps)

    dec = params["dec"]
    d0 = dec["layers"][0]["e1"]
    ctab0 = jnp.concatenate([cinit, jnp.zeros((N, PW - 3), _f32)], axis=1)
    hd, ad, bd = _dec_init(
        z, bcol, ctab0, dec["emb_in"]["W"], _row(dec["emb_in"]["b"]),
        d0["W"][:HID], _row(d0["b"]), d0["W"][HID:2 * HID])
    h_dec, ctab = _egnn_layers(
        dec["layers"], hd, ad, bd, ctab0, srcg, dstg, dsts,
        dec["emb_out"]["W"], dec["emb_out"]["b"])

    atom_pred, energy_pred = _final(
        h_dec, bcol,
        params["atom1"]["W"], _row(params["atom1"]["b"]),
        params["atom2"]["W"], _row(params["atom2"]["b"]),
        params["en1"]["W"], _row(params["en1"]["b"]),
        params["en2"]["W"], _row(params["en2"]["b"]))

    coords_out = ctab[:, 0:3]
    return (atom_pred, coords_out, energy_pred, mu, logvar)
